# baseline (speedup 1.0000x reference)
import jax
import jax.numpy as jnp
from jax.experimental import pallas as pl
from jax.experimental.pallas import tpu as pltpu

_ISQRT2 = 0.7071067811865476

_B = 16
_V = 256
_FD = 32
_H = 256
_NA = 16
_NB = 4
_DB = _FD // _NB
_NC = 8
_VC = _V // _NC


def _fused_body(v_ref, wg_ref, bg_ref, w1_ref, b1_ref, w2_ref, b2_ref,
                out_ref, carry_ref, pooledT_ref, h1_ref):
    i = pl.program_id(0)

    @pl.when(i < _NA)
    def _phase_a():
        @pl.when(i == 0)
        def _init():
            carry_ref[...] = jnp.zeros_like(carry_ref)

        v = v_ref[0]
        vt = jnp.swapaxes(v, 1, 2)
        h = jnp.dot(vt.reshape(_V * _FD, _FD), wg_ref[...],
                    preferred_element_type=jnp.float32)
        h = h.reshape(_V, _FD, _FD)
        hprev = jnp.concatenate([carry_ref[...], h[:-1]], axis=0)
        carry_ref[...] = h[-1:]
        g = jax.lax.broadcasted_iota(jnp.int32, (_V, 1, 1), 0) + i * _V
        alpha = jnp.where(g == 0, 0.0, jnp.where(g == 1, _ISQRT2, 0.5))
        beta = jnp.where(g == 0, 1.0, 0.5)
        mixed = (alpha.astype(jnp.float32) * hprev
                 + beta.astype(jnp.float32) * h)
        pooled = jnp.max(mixed, axis=1) + bg_ref[...][None, :]
        pooledT_ref[:, i, :] = pooled.T

    @pl.when(jnp.logical_and(i >= _NA, i < _NA + _NB))
    def _phase_b():
        @pl.when(i == _NA)
        def _init():
            h1_ref[...] = jnp.zeros_like(h1_ref)

        pb = i - _NA
        acc = h1_ref[...]
        for k in range(_DB):
            acc += jnp.dot(pooledT_ref[pb * _DB + k], w1_ref[:, k, :],
                           preferred_element_type=jnp.float32)
        h1_ref[...] = acc

        @pl.when(i == _NA + _NB - 1)
        def _relu():
            h1_ref[...] = jnp.maximum(h1_ref[...] + b1_ref[...][None, :],
                                      0.0)

    @pl.when(i >= _NA + _NB)
    def _phase_c():
        o = jnp.dot(h1_ref[...], w2_ref[...],
                    preferred_element_type=jnp.float32)
        o = jax.nn.sigmoid(o + b2_ref[...][None, :])
        for v in range(_VC):
            out_ref[:, v, :] = o[:, v * _FD:(v + 1) * _FD]


def kernel(videos, W_gcn, b_gcn, W1, b1, W2, b2):
    B, V, F, D = videos.shape
    W1v = W1.reshape(V, D, _H)

    def _bidx(i):
        return jnp.clip(i - _NA, 0, _NB - 1)

    def _cidx(i):
        return jnp.clip(i - _NA - _NB, 0, _NC - 1)

    out = pl.pallas_call(
        _fused_body,
        grid=(_NA + _NB + _NC,),
        in_specs=[
            pl.BlockSpec((1, V, F, D),
                         lambda i: (jnp.minimum(i, _NA - 1), 0, 0, 0)),
            pl.BlockSpec((F, D), lambda i: (0, 0)),
            pl.BlockSpec((D,), lambda i: (0,)),
            pl.BlockSpec((V, _DB, _H), lambda i: (0, _bidx(i), 0)),
            pl.BlockSpec((_H,), lambda i: (0,)),
            pl.BlockSpec((_H, _VC * F), lambda i: (0, _cidx(i))),
            pl.BlockSpec((_VC * F,), lambda i: (_cidx(i),)),
        ],
        out_specs=pl.BlockSpec((B, _VC, F), lambda i: (0, _cidx(i), 0)),
        out_shape=jax.ShapeDtypeStruct((B, V, F), jnp.float32),
        scratch_shapes=[
            pltpu.VMEM((1, _FD, _FD), jnp.float32),
            pltpu.VMEM((_FD, _B, _V), jnp.float32),
            pltpu.VMEM((_B, _H), jnp.float32),
        ],
    )(videos, W_gcn, b_gcn, W1v, b1, W2, b2)
    return out


# speedup vs baseline: 1.2554x; 1.2554x over previous
import jax
import jax.numpy as jnp
from jax.experimental import pallas as pl
from jax.experimental.pallas import tpu as pltpu

_ISQRT2 = 0.7071067811865476

_B = 16
_V = 256
_FD = 32
_H = 256
_NA = 16
_NB = 4
_DB = _FD // _NB
_NC = 8
_VC = _V // _NC


def _fused_body(v_ref, wg_ref, bg_ref, w1_ref, b1_ref, w2_ref, b2_ref,
                out_ref, carry_ref, pooledT_ref, h1_ref):
    i = pl.program_id(0)

    @pl.when(i < _NA)
    def _phase_a():
        @pl.when(i == 0)
        def _init():
            carry_ref[...] = jnp.zeros_like(carry_ref)

        v = v_ref[...]
        vt = jnp.swapaxes(v, 1, 2)
        h = jnp.dot(vt.reshape(_V * _FD, _FD), wg_ref[...],
                    preferred_element_type=jnp.float32)
        h = h.reshape(_V, _FD, _FD)
        hprev = jnp.concatenate([carry_ref[...], h[:-1]], axis=0)
        carry_ref[...] = h[-1:]
        g = jax.lax.broadcasted_iota(jnp.int32, (_V, 1, 1), 0) + i * _V
        alpha = jnp.where(g == 0, 0.0, jnp.where(g == 1, _ISQRT2, 0.5))
        beta = jnp.where(g == 0, 1.0, 0.5)
        mixed = (alpha.astype(jnp.float32) * hprev
                 + beta.astype(jnp.float32) * h)
        pooled = jnp.max(mixed, axis=1) + bg_ref[...][None, :]
        pooledT_ref[:, i, :] = pooled.T

    @pl.when(jnp.logical_and(i >= _NA, i < _NA + _NB))
    def _phase_b():
        @pl.when(i == _NA)
        def _init():
            h1_ref[...] = jnp.zeros_like(h1_ref)

        pb = i - _NA
        acc = h1_ref[...]
        for k in range(_DB):
            acc += jnp.dot(pooledT_ref[pb * _DB + k], w1_ref[:, k, :],
                           preferred_element_type=jnp.float32)
        h1_ref[...] = acc

        @pl.when(i == _NA + _NB - 1)
        def _relu():
            h1_ref[...] = jnp.maximum(h1_ref[...] + b1_ref[...][None, :],
                                      0.0)

    @pl.when(i >= _NA + _NB)
    def _phase_c():
        o = jnp.dot(h1_ref[...], w2_ref[...],
                    preferred_element_type=jnp.float32)
        o = jax.nn.sigmoid(o + b2_ref[...][None, :])
        for v in range(_VC):
            out_ref[:, v, :] = o[:, v * _FD:(v + 1) * _FD]


def kernel(videos, W_gcn, b_gcn, W1, b1, W2, b2):
    B, V, F, D = videos.shape
    v2 = videos.reshape(B * V, F, D)
    W1v = W1.reshape(V, D, _H)

    def _bidx(i):
        return jnp.clip(i - _NA, 0, _NB - 1)

    def _cidx(i):
        return jnp.clip(i - _NA - _NB, 0, _NC - 1)

    out = pl.pallas_call(
        _fused_body,
        grid=(_NA + _NB + _NC,),
        in_specs=[
            pl.BlockSpec((V, F, D),
                         lambda i: (jnp.minimum(i, _NA - 1), 0, 0)),
            pl.BlockSpec((F, D), lambda i: (0, 0)),
            pl.BlockSpec((D,), lambda i: (0,)),
            pl.BlockSpec((V, _DB, _H), lambda i: (0, _bidx(i), 0)),
            pl.BlockSpec((_H,), lambda i: (0,)),
            pl.BlockSpec((_H, _VC * F), lambda i: (0, _cidx(i))),
            pl.BlockSpec((_VC * F,), lambda i: (_cidx(i),)),
        ],
        out_specs=pl.BlockSpec((B, _VC, F), lambda i: (0, _cidx(i), 0)),
        out_shape=jax.ShapeDtypeStruct((B, V, F), jnp.float32),
        scratch_shapes=[
            pltpu.VMEM((1, _FD, _FD), jnp.float32),
            pltpu.VMEM((_FD, _B, _V), jnp.float32),
            pltpu.VMEM((_B, _H), jnp.float32),
        ],
    )(v2, W_gcn, b_gcn, W1v, b1, W2, b2)
    return out
